# Initial kernel scaffold; baseline (speedup 1.0000x reference)
import jax
import jax.numpy as jnp
from jax.experimental import pallas as pl


def kernel(nodes, history_uv, history_r, feat_table, r_table, W_gv, b_gv, W1, b1):
    raise NotImplementedError("write your pallas kernel here")


# speedup vs baseline: 5.4524x; 5.4524x over previous
import functools

import jax
import jax.numpy as jnp
from jax import lax
from jax.experimental import pallas as pl
from jax.experimental.pallas import tpu as pltpu
from jax.experimental.pallas import tpu_sc as plsc

NUM_NODES = 100000
EMBED_DIM = 128
BATCH = 16384
HIST_LEN = 32

NC, NS = 2, 16
NW = NC * NS
BPW = BATCH // NW
CB = 8
ROWS_PER_CHUNK = CB * HIST_LEN
NCHUNKS = BPW // CB
GSZ = 128


def _proj_body(feat_ref, wg_ref, w1_ref, fp_ref, sp_ref):
    x = feat_ref[...]
    fp_ref[...] = jnp.dot(x, wg_ref[...], preferred_element_type=jnp.float32)
    sp_ref[...] = jnp.dot(x, w1_ref[...], preferred_element_type=jnp.float32)


def _project_tables(feat_table, wg_top, w1_top):
    blk = 2000
    grid = NUM_NODES // blk
    return pl.pallas_call(
        _proj_body,
        grid=(grid,),
        in_specs=[
            pl.BlockSpec((blk, EMBED_DIM), lambda i: (i, 0)),
            pl.BlockSpec((EMBED_DIM, EMBED_DIM), lambda i: (0, 0)),
            pl.BlockSpec((EMBED_DIM, EMBED_DIM), lambda i: (0, 0)),
        ],
        out_specs=[
            pl.BlockSpec((blk, EMBED_DIM), lambda i: (i, 0)),
            pl.BlockSpec((blk, EMBED_DIM), lambda i: (i, 0)),
        ],
        out_shape=[
            jax.ShapeDtypeStruct((NUM_NODES, EMBED_DIM), jnp.float32),
            jax.ShapeDtypeStruct((NUM_NODES, EMBED_DIM), jnp.float32),
        ],
    )(feat_table, wg_top, w1_top)


def _rproj_body(r_ref, wg_ref, b_ref, out_ref):
    out_ref[...] = (
        jnp.dot(r_ref[...], wg_ref[...], preferred_element_type=jnp.float32)
        + b_ref[...]
    )


def _project_ratings(r_pad, wg_bot, b_gv):
    return pl.pallas_call(
        _rproj_body,
        out_shape=jax.ShapeDtypeStruct((8, EMBED_DIM), jnp.float32),
    )(r_pad, wg_bot, b_gv.reshape(1, EMBED_DIM))


def _final_body(self_ref, neigh_ref, w_ref, b_ref, out_ref):
    t = jnp.dot(neigh_ref[...], w_ref[...], preferred_element_type=jnp.float32)
    out_ref[...] = jnp.maximum(self_ref[...] + t + b_ref[...], 0.0)


def _final_combine(self_rows, neigh, w1_bot, b1):
    blk = 512
    return pl.pallas_call(
        _final_body,
        grid=(BATCH // blk,),
        in_specs=[
            pl.BlockSpec((blk, EMBED_DIM), lambda i: (i, 0)),
            pl.BlockSpec((blk, EMBED_DIM), lambda i: (i, 0)),
            pl.BlockSpec((EMBED_DIM, EMBED_DIM), lambda i: (0, 0)),
            pl.BlockSpec((1, EMBED_DIM), lambda i: (0, 0)),
        ],
        out_specs=pl.BlockSpec((blk, EMBED_DIM), lambda i: (i, 0)),
        out_shape=jax.ShapeDtypeStruct((BATCH, EMBED_DIM), jnp.float32),
    )(self_rows, neigh, w1_bot, b1.reshape(1, EMBED_DIM))


def _sc_body(feat_proj, self_proj, uv_flat, r_flat, rproj_hbm, nodes,
             neigh_out, self_out,
             idx00, idx01, idx10, idx11, r0, r1, rows0, rows1,
             rproj_v, stage, sem0, sem1):
    cid = lax.axis_index("c")
    sid = lax.axis_index("s")
    w = sid * NC + cid
    row_base = w * (BPW * HIST_LEN)
    b_base = w * BPW

    pltpu.sync_copy(rproj_hbm, rproj_v)

    def issue(c, idx_a, idx_b, r_ref, rows_ref, sem):
        base = row_base + c * ROWS_PER_CHUNK
        pltpu.sync_copy(uv_flat.at[pl.ds(base, GSZ)], idx_a)
        pltpu.sync_copy(uv_flat.at[pl.ds(base + GSZ, GSZ)], idx_b)
        pltpu.sync_copy(r_flat.at[pl.ds(base, ROWS_PER_CHUNK)], r_ref)
        pltpu.async_copy(feat_proj.at[idx_a], rows_ref.at[pl.ds(0, GSZ)], sem)
        pltpu.async_copy(feat_proj.at[idx_b], rows_ref.at[pl.ds(GSZ, GSZ)], sem)

    def drain(idx_a, idx_b, rows_ref, sem):
        pltpu.make_async_copy(
            feat_proj.at[idx_a], rows_ref.at[pl.ds(0, GSZ)], sem).wait()
        pltpu.make_async_copy(
            feat_proj.at[idx_b], rows_ref.at[pl.ds(GSZ, GSZ)], sem).wait()

    inv = jnp.float32(1.0 / HIST_LEN)

    def compute(c, r_ref, rows_ref):
        def kbody(k, carry):
            acc = [jnp.zeros((16,), jnp.float32) for _ in range(8)]
            for g in range(HIST_LEN // 16):
                rvec = r_ref[pl.ds(k * HIST_LEN + g * 16, 16)]
                for m in range(16):
                    row = k * HIST_LEN + g * 16 + m
                    roff = rvec[m] * EMBED_DIM
                    for j in range(8):
                        v = rows_ref[row, pl.ds(j * 16, 16)]
                        rv = rproj_v[pl.ds(roff + j * 16, 16)]
                        acc[j] = acc[j] + jnp.maximum(v + rv, 0.0)
            for j in range(8):
                stage[pl.ds(k * EMBED_DIM + j * 16, 16)] = acc[j] * inv
            return carry

        lax.fori_loop(0, CB, kbody, 0)
        pltpu.sync_copy(
            stage,
            neigh_out.at[pl.ds((b_base + c * CB) * EMBED_DIM,
                               CB * EMBED_DIM)])

    issue(0, idx00, idx01, r0, rows0, sem0)
    issue(1, idx10, idx11, r1, rows1, sem1)

    def chunk_pair(i, carry):
        c0 = 2 * i
        c1 = 2 * i + 1
        drain(idx00, idx01, rows0, sem0)
        compute(c0, r0, rows0)

        @pl.when(c0 + 2 < NCHUNKS)
        def _():
            issue(c0 + 2, idx00, idx01, r0, rows0, sem0)

        drain(idx10, idx11, rows1, sem1)
        compute(c1, r1, rows1)

        @pl.when(c1 + 2 < NCHUNKS)
        def _():
            issue(c1 + 2, idx10, idx11, r1, rows1, sem1)

        return carry

    lax.fori_loop(0, NCHUNKS // 2, chunk_pair, 0)

    for t in range(BPW // GSZ):
        pltpu.sync_copy(nodes.at[pl.ds(b_base + t * GSZ, GSZ)], idx00)
        pltpu.async_copy(
            self_proj.at[idx00], rows0.at[pl.ds(0, GSZ)], sem0).wait()
        pltpu.sync_copy(rows0.at[pl.ds(0, GSZ)],
                        self_out.at[pl.ds(b_base + t * GSZ, GSZ)])


@functools.partial(jax.jit, static_argnames=())
def _sc_gather_pool(feat_proj, self_proj, uv_flat, r_flat, rproj, nodes):
    mesh = plsc.VectorSubcoreMesh(core_axis_name="c", subcore_axis_name="s")
    f = pl.kernel(
        _sc_body,
        out_type=[
            jax.ShapeDtypeStruct((BATCH * EMBED_DIM,), jnp.float32),
            jax.ShapeDtypeStruct((BATCH, EMBED_DIM), jnp.float32),
        ],
        mesh=mesh,
        scratch_types=[
            pltpu.VMEM((GSZ,), jnp.int32),
            pltpu.VMEM((GSZ,), jnp.int32),
            pltpu.VMEM((GSZ,), jnp.int32),
            pltpu.VMEM((GSZ,), jnp.int32),
            pltpu.VMEM((ROWS_PER_CHUNK,), jnp.int32),
            pltpu.VMEM((ROWS_PER_CHUNK,), jnp.int32),
            pltpu.VMEM((ROWS_PER_CHUNK, EMBED_DIM), jnp.float32),
            pltpu.VMEM((ROWS_PER_CHUNK, EMBED_DIM), jnp.float32),
            pltpu.VMEM((8 * EMBED_DIM,), jnp.float32),
            pltpu.VMEM((CB * EMBED_DIM,), jnp.float32),
            pltpu.SemaphoreType.DMA,
            pltpu.SemaphoreType.DMA,
        ],
    )
    return f(feat_proj, self_proj, uv_flat, r_flat, rproj, nodes)


def kernel(nodes, history_uv, history_r, feat_table, r_table,
           W_gv, b_gv, W1, b1):
    uv_flat = history_uv.reshape(-1).astype(jnp.int32)
    r_flat = history_r.reshape(-1).astype(jnp.int32)
    nodes32 = nodes.astype(jnp.int32)

    wg_top, wg_bot = W_gv[:EMBED_DIM], W_gv[EMBED_DIM:]
    w1_top, w1_bot = W1[:EMBED_DIM], W1[EMBED_DIM:]

    feat_proj, self_proj = _project_tables(feat_table, wg_top, w1_top)
    r_pad = jnp.concatenate(
        [r_table, jnp.zeros((2, EMBED_DIM), jnp.float32)], axis=0)
    rproj = _project_ratings(r_pad, wg_bot, b_gv)

    neigh_flat, self_rows = _sc_gather_pool(
        feat_proj, self_proj, uv_flat, r_flat,
        rproj.reshape(8 * EMBED_DIM), nodes32)
    neigh = neigh_flat.reshape(BATCH, EMBED_DIM)

    return _final_combine(self_rows, neigh, w1_bot, b1)
